# baseline (speedup 1.0000x reference)
import functools

import jax
import jax.numpy as jnp
from jax import lax
from jax.experimental import pallas as pl
from jax.experimental.pallas import tpu as pltpu
from jax.experimental.pallas import tpu_sc as plsc

_NC, _NS = 2, 16
_CHUNK = 800
_ROWS = 3200


def _sc_gather(wv, idx4):
    num_idx = idx4.shape[0]
    dw = wv.shape[1]
    nw = _NC * _NS
    b_per_w = num_idx // nw
    mesh = plsc.VectorSubcoreMesh(core_axis_name="c", subcore_axis_name="s")

    @functools.partial(
        pl.kernel,
        mesh=mesh,
        out_type=jax.ShapeDtypeStruct((num_idx, dw), wv.dtype),
        scratch_types=[
            pltpu.VMEM((_CHUNK,), jnp.int32),
            pltpu.VMEM((_CHUNK, dw), jnp.float32),
            pltpu.SemaphoreType.DMA,
        ],
    )
    def gather_kernel(w_hbm, i_hbm, o_hbm, idx_v, rows_v, sem):
        wid = lax.axis_index("s") * _NC + lax.axis_index("c")
        base = wid * b_per_w

        @pl.loop(0, b_per_w, step=_CHUNK)
        def _(off):
            pltpu.sync_copy(i_hbm.at[pl.ds(base + off, _CHUNK)], idx_v)
            pltpu.async_copy(w_hbm.at[idx_v], rows_v, sem).wait()
            pltpu.sync_copy(rows_v, o_hbm.at[pl.ds(base + off, _CHUNK)])

    return gather_kernel(wv, idx4)


_BB = 256


def _select_normalize(g, qw, b, h, d):
    n, dw = g.shape
    nsub = dw // d
    rows = _BB * h

    def body(g_ref, q_ref, o_ref):
        gb = g_ref[...]
        qb = q_ref[...]
        acc = jnp.zeros((rows, d), jnp.float32)
        for k in range(nsub):
            acc = jnp.where(qb == float(k), gb[:, k * d:(k + 1) * d], acc)
        s = jax.lax.dot_general(
            acc * acc, jnp.ones((d, 1), jnp.float32),
            (((1,), (0,)), ((), ())), preferred_element_type=jnp.float32)
        acc = acc / jnp.maximum(jnp.sqrt(s), 1e-12)
        for p in range(_BB):
            o_ref[p, :, :] = acc[p * h:(p + 1) * h, :]

    return pl.pallas_call(
        body,
        grid=(b // _BB,),
        in_specs=[
            pl.BlockSpec((rows, dw), lambda i: (i, 0)),
            pl.BlockSpec((rows, d), lambda i: (i, 0)),
        ],
        out_specs=pl.BlockSpec((_BB, h, d), lambda i: (i, 0, 0)),
        out_shape=jax.ShapeDtypeStruct((b, h, d), jnp.float32),
    )(g, qw)


def kernel(x, weight):
    b, h = x.shape
    n, d = weight.shape
    num_idx = b * h
    xi = x.astype(jnp.int32).reshape(num_idx)
    idx4 = xi // 4
    qw = jnp.broadcast_to(
        (xi % 4).astype(jnp.float32).reshape(num_idx, 1), (num_idx, d))
    wflat = jax.lax.optimization_barrier(weight.reshape(n * d))
    wv = wflat.reshape(n // 4, 4 * d)
    g = _sc_gather(wv, idx4)
    return _select_normalize(g, qw, b, h, d)
